# baseline (speedup 1.0000x reference)
import jax
import jax.numpy as jnp
from jax.experimental import pallas as pl
from jax.experimental.pallas import tpu as pltpu

_LN_EPS = 1e-5


def _ln_rows(x):
    mu = jnp.mean(x, axis=1, keepdims=True)
    var = jnp.mean((x - mu) ** 2, axis=1, keepdims=True)
    return (x - mu) * jax.lax.rsqrt(var + _LN_EPS)


def _node_kernel(hn_ref, nx_ref,
                 lw1_ref, lb1_ref, lg_ref, lbe_ref, lw2_ref, lb2_ref,
                 rw1_ref, rb1_ref, rg_ref, rbe_ref, rw2_ref, rb2_ref,
                 wnl_ref, wnr_ref, iw1_ref, wb_ref,
                 gwe_ref, gwl_ref, gwr_ref, gwx_ref,
                 ib1_ref, gb1_ref,
                 tl_ref, tr_ref, wc_ref):
    hn = hn_ref[...]

    def mlp(w1, b1, g, be, w2, b2):
        t = jnp.dot(hn, w1[...], preferred_element_type=jnp.float32) + b1[...]
        t = jnp.maximum(_ln_rows(t) * g[...] + be[...], 0.0)
        return jnp.dot(t, w2[...], preferred_element_type=jnp.float32) + b2[...]

    left = mlp(lw1_ref, lb1_ref, lg_ref, lbe_ref, lw2_ref, lb2_ref)
    right = mlp(rw1_ref, rb1_ref, rg_ref, rbe_ref, rw2_ref, rb2_ref)

    lwn = jnp.dot(left, wnl_ref[...], preferred_element_type=jnp.float32)
    v_l = jnp.dot(lwn, iw1_ref[...],
                  preferred_element_type=jnp.float32) + ib1_ref[...]
    g_l = (jnp.dot(left, gwl_ref[...], preferred_element_type=jnp.float32)
           + jnp.dot(nx_ref[...], gwx_ref[...], preferred_element_type=jnp.float32)
           + gb1_ref[...])
    tl_ref[...] = jnp.concatenate([v_l, g_l], axis=1).astype(jnp.bfloat16)

    rwn = jnp.dot(right, wnr_ref[...], preferred_element_type=jnp.float32)
    v_r = jnp.dot(rwn, iw1_ref[...], preferred_element_type=jnp.float32)
    g_r = jnp.dot(right, gwr_ref[...], preferred_element_type=jnp.float32)
    tr_ref[...] = jnp.concatenate([v_r, g_r], axis=1).astype(jnp.bfloat16)

    wm = jnp.dot(wb_ref[...], iw1_ref[...], preferred_element_type=jnp.float32)
    wc_ref[...] = jnp.concatenate([wm, gwe_ref[...]], axis=1).astype(jnp.bfloat16)


def _edge_kernel(he_ref, rvd_ref, el_ref, er_ref,
                 tl_ref, tr_ref, wc_ref, slab_ref, out_ref):
    k = pl.program_id(1)

    @pl.when(k == 0)
    def _init():
        out_ref[...] = jnp.zeros_like(out_ref)

    n_pad = tl_ref.shape[0]
    n_hi = n_pad // 128
    te = he_ref.shape[0]
    el = el_ref[...]
    er = er_ref[...]

    iota = jax.lax.broadcasted_iota(jnp.int32, (te, n_pad), 1)
    oh_l = (iota == el).astype(jnp.bfloat16)
    oh_r = (iota == er).astype(jnp.bfloat16)

    proj = jnp.dot(he_ref[...].astype(jnp.bfloat16), wc_ref[...],
                   preferred_element_type=jnp.float32)
    gl = jnp.dot(oh_l, tl_ref[...], preferred_element_type=jnp.float32)
    gr = jnp.dot(oh_r, tr_ref[...], preferred_element_type=jnp.float32)
    xt = jnp.transpose(proj + gl + gr)

    sl = slab_ref[...]

    def ln_cols(x):
        mu = jnp.mean(x, axis=0, keepdims=True)
        var = jnp.mean((x - mu) ** 2, axis=0, keepdims=True)
        return (x - mu) * jax.lax.rsqrt(var + _LN_EPS)

    ih = jnp.maximum(ln_cols(xt[:128]) * sl[:, 0:1] + sl[:, 1:2], 0.0)
    inter = (jnp.sum(ih * sl[:, 2:3], axis=0, keepdims=True)
             + sl[0:1, 6:7])

    gh = jnp.maximum(ln_cols(xt[128:160]) * sl[:32, 3:4] + sl[:32, 4:5], 0.0)
    gate = (jnp.sum(gh * sl[:32, 5:6], axis=0, keepdims=True)
            + sl[1:2, 6:7])

    weight = inter * jax.nn.sigmoid(gate)

    rvd = jnp.transpose(rvd_ref[...])
    dist = rvd[4:5, :]
    coef = 5.0 / ((dist + 1e-6) * (dist + 5.0))
    force = jnp.transpose(rvd[:4] * (weight * coef))

    a_mask = jax.lax.broadcasted_iota(jnp.int32, (te, n_hi), 1) == (el >> 7)
    b_hot = (jax.lax.broadcasted_iota(jnp.int32, (te, 128), 1) == (el & 127)
             ).astype(jnp.float32)
    lhs = jnp.concatenate(
        [jnp.where(a_mask, force[:, c:c + 1], 0.0) for c in range(3)],
        axis=1)
    part = jax.lax.dot_general(lhs, b_hot, (((0,), (0,)), ((), ())),
                               preferred_element_type=jnp.float32)
    out_ref[...] += part


def _fin_kernel(part_ref, hn_ref, nx_ref, pw1h_ref, pw1x_ref, slab_ref, out_ref):
    delta = part_ref[0]
    for c in range(1, part_ref.shape[0]):
        delta = delta + part_ref[c]

    sl = slab_ref[...]
    norm = jnp.sqrt(jnp.sum(delta * delta, axis=1, keepdims=True))
    ph = (jnp.dot(hn_ref[...], pw1h_ref[...], preferred_element_type=jnp.float32)
          + jnp.dot(nx_ref[...], pw1x_ref[...], preferred_element_type=jnp.float32)
          + norm * sl[0:1, :] + sl[1:2, :])
    ph = jnp.maximum(_ln_rows(ph) * sl[2:3, :] + sl[3:4, :], 0.0)
    scale = jax.nn.sigmoid(jnp.sum(ph * sl[4:5, :], axis=1, keepdims=True)
                           + sl[5:6, 0:1])
    out_ref[...] = delta * scale


def kernel(h_node, h_edge, edge_index, relative_vec, distance, node_extra,
           lw1, lb1, lg, lbe, lw2, lb2,
           rw1, rb1, rg, rbe, rw2, rb2,
           wb, wn_l, wn_r,
           iw1, ib1, ig, ibe, iw2, ib2,
           gw1_e, gw1_l, gw1_r, gw1_x, gb1, gg, gbe, gw2, gb2,
           pw1_h, pw1_x, pw1_n, pb1, pg, pbe, pw2, pb2,
           *, edge_tile=1024, num_splits=2):
    N, D = h_node.shape
    E, De = h_edge.shape
    N_pad = -(-N // 128) * 128

    f32 = lambda a: jnp.asarray(a, jnp.float32)
    e_left = edge_index[0].astype(jnp.int32)
    e_right = edge_index[1].astype(jnp.int32)

    h_node = f32(h_node)
    nx = f32(node_extra)

    full = lambda a: pl.BlockSpec(a.shape, lambda: (0,) * a.ndim)
    node_ins = [h_node, nx,
                f32(lw1), f32(lb1), f32(lg), f32(lbe), f32(lw2), f32(lb2),
                f32(rw1), f32(rb1), f32(rg), f32(rbe), f32(rw2), f32(rb2),
                f32(wn_l), f32(wn_r), f32(iw1), f32(wb),
                f32(gw1_e), f32(gw1_l), f32(gw1_r), f32(gw1_x),
                f32(ib1), f32(gb1)]
    table_l, table_r, wcomb = pl.pallas_call(
        _node_kernel,
        out_shape=(jax.ShapeDtypeStruct((N, D + 32), jnp.bfloat16),
                   jax.ShapeDtypeStruct((N, D + 32), jnp.bfloat16),
                   jax.ShapeDtypeStruct((De, D + 32), jnp.bfloat16)),
        in_specs=[full(a) for a in node_ins],
        out_specs=(pl.BlockSpec((N, D + 32), lambda: (0, 0)),
                   pl.BlockSpec((N, D + 32), lambda: (0, 0)),
                   pl.BlockSpec((De, D + 32), lambda: (0, 0))),
    )(*node_ins)

    if N_pad > N:
        znp = lambda a: jnp.pad(a, ((0, N_pad - N), (0, 0)))
        table_l, table_r = znp(table_l), znp(table_r)

    he = f32(h_edge)
    rvd = jnp.concatenate(
        [f32(relative_vec), jnp.zeros((E, 1), jnp.float32),
         f32(distance)[:, None], jnp.zeros((E, 3), jnp.float32)], axis=1)

    edge_tile = max(128, (edge_tile // 128) * 128)
    per_split = -(-E // num_splits)
    edge_tile = min(edge_tile, -(-per_split // 128) * 128)
    chunk = num_splits * edge_tile
    E_pad = -(-E // chunk) * chunk
    nk = E_pad // chunk
    pad = E_pad - E
    if pad:
        he = jnp.pad(he, ((0, pad), (0, 0)))
        rvd = jnp.pad(rvd, ((0, pad), (0, 0)))
        e_left = jnp.pad(e_left, (0, pad))
        e_right = jnp.pad(e_right, (0, pad))
    el_col = e_left[:, None]
    er_col = e_right[:, None]

    padc = lambda r: jnp.pad(f32(r).reshape(-1, 1), ((0, 128 - r.size), (0, 0)))
    slab1 = jnp.concatenate(
        [padc(ig), padc(ibe), padc(iw2),
         padc(gg), padc(gbe), padc(gw2),
         padc(jnp.concatenate([f32(ib2).reshape(1, 1),
                               f32(gb2).reshape(1, 1)], axis=0)),
         jnp.zeros((128, 1), jnp.float32)], axis=1)

    estream = lambda cols: pl.BlockSpec((edge_tile, cols),
                                        lambda c, k: (c * nk + k, 0))
    resident = lambda a: pl.BlockSpec(a.shape, lambda c, k: (0, 0))

    partial = pl.pallas_call(
        _edge_kernel,
        out_shape=jax.ShapeDtypeStruct(
            (num_splits, 3 * (N_pad // 128), 128), jnp.float32),
        grid=(num_splits, nk),
        in_specs=[estream(De), estream(8),
                  estream(1), estream(1),
                  resident(table_l), resident(table_r),
                  resident(wcomb), resident(slab1)],
        out_specs=pl.BlockSpec((None, 3 * (N_pad // 128), 128),
                               lambda c, k: (c, 0, 0)),
        compiler_params=pltpu.CompilerParams(
            dimension_semantics=("parallel", "arbitrary"),
            vmem_limit_bytes=64 * 1024 * 1024),
    )(he, rvd, el_col, er_col, table_l, table_r, wcomb, slab1)

    delta_nm = partial.reshape(num_splits, 3, N_pad).transpose(0, 2, 1)

    npad_n = N_pad - N
    hn_p = jnp.pad(h_node, ((0, npad_n), (0, 0))) if npad_n else h_node
    nx_p = jnp.pad(nx, ((0, npad_n), (0, 0))) if npad_n else nx

    pad128 = lambda r: jnp.pad(f32(r).reshape(1, -1),
                               ((0, 0), (0, 128 - r.size)))
    slab2 = jnp.concatenate(
        [f32(pw1_n), f32(pb1), f32(pg), f32(pbe),
         f32(pw2).reshape(1, -1), pad128(f32(pb2).reshape(1, 1)),
         jnp.zeros((2, 128), jnp.float32)], axis=0)

    fin_ins = [delta_nm, hn_p, nx_p, f32(pw1_h), f32(pw1_x), slab2]
    out = pl.pallas_call(
        _fin_kernel,
        out_shape=jax.ShapeDtypeStruct((N_pad, 3), jnp.float32),
        in_specs=[full(a) for a in fin_ins],
        out_specs=pl.BlockSpec((N_pad, 3), lambda: (0, 0)),
    )(*fin_ins)

    return out[:N]


# speedup vs baseline: 1.0800x; 1.0800x over previous
import jax
import jax.numpy as jnp
from jax.experimental import pallas as pl
from jax.experimental.pallas import tpu as pltpu

_LN_EPS = 1e-5


def _ln_rows(x):
    mu = jnp.mean(x, axis=1, keepdims=True)
    var = jnp.mean((x - mu) ** 2, axis=1, keepdims=True)
    return (x - mu) * jax.lax.rsqrt(var + _LN_EPS)


def _node_kernel(hn_ref, nx_ref,
                 lw1_ref, lb1_ref, lg_ref, lbe_ref, lw2_ref, lb2_ref,
                 rw1_ref, rb1_ref, rg_ref, rbe_ref, rw2_ref, rb2_ref,
                 wnl_ref, wnr_ref, iw1_ref, wb_ref,
                 gwe_ref, gwl_ref, gwr_ref, gwx_ref,
                 ib1_ref, gb1_ref,
                 tl_ref, tr_ref, wc_ref):
    hn = hn_ref[...]

    def mlp(w1, b1, g, be, w2, b2):
        t = jnp.dot(hn, w1[...], preferred_element_type=jnp.float32) + b1[...]
        t = jnp.maximum(_ln_rows(t) * g[...] + be[...], 0.0)
        return jnp.dot(t, w2[...], preferred_element_type=jnp.float32) + b2[...]

    left = mlp(lw1_ref, lb1_ref, lg_ref, lbe_ref, lw2_ref, lb2_ref)
    right = mlp(rw1_ref, rb1_ref, rg_ref, rbe_ref, rw2_ref, rb2_ref)

    lwn = jnp.dot(left, wnl_ref[...], preferred_element_type=jnp.float32)
    v_l = jnp.dot(lwn, iw1_ref[...],
                  preferred_element_type=jnp.float32) + ib1_ref[...]
    g_l = (jnp.dot(left, gwl_ref[...], preferred_element_type=jnp.float32)
           + jnp.dot(nx_ref[...], gwx_ref[...], preferred_element_type=jnp.float32)
           + gb1_ref[...])
    tl_ref[...] = jnp.concatenate([v_l, g_l], axis=1).astype(jnp.bfloat16)

    rwn = jnp.dot(right, wnr_ref[...], preferred_element_type=jnp.float32)
    v_r = jnp.dot(rwn, iw1_ref[...], preferred_element_type=jnp.float32)
    g_r = jnp.dot(right, gwr_ref[...], preferred_element_type=jnp.float32)
    tr_ref[...] = jnp.concatenate([v_r, g_r], axis=1).astype(jnp.bfloat16)

    wm = jnp.dot(wb_ref[...], iw1_ref[...], preferred_element_type=jnp.float32)
    wc_ref[...] = jnp.concatenate([wm, gwe_ref[...]], axis=1).astype(jnp.bfloat16)


def _edge_kernel(he_ref, rvd_ref, el_ref, er_ref,
                 tl_ref, tr_ref, wc_ref, slab_ref, out_ref):
    k = pl.program_id(1)

    @pl.when(k == 0)
    def _init():
        out_ref[...] = jnp.zeros_like(out_ref)

    n_pad = tl_ref.shape[0]
    n_hi = n_pad // 128
    te = he_ref.shape[0]
    el = el_ref[...]
    er = er_ref[...]

    iota = jax.lax.broadcasted_iota(jnp.int32, (te, n_pad), 1)
    oh_l = (iota == el).astype(jnp.bfloat16)
    oh_r = (iota == er).astype(jnp.bfloat16)

    proj = jnp.dot(he_ref[...].astype(jnp.bfloat16), wc_ref[...],
                   preferred_element_type=jnp.float32)
    gl = jnp.dot(oh_l, tl_ref[...], preferred_element_type=jnp.float32)
    gr = jnp.dot(oh_r, tr_ref[...], preferred_element_type=jnp.float32)
    xt = jnp.transpose(proj + gl + gr)

    sl = slab_ref[...]

    def ln_cols(x):
        mu = jnp.mean(x, axis=0, keepdims=True)
        var = jnp.mean((x - mu) ** 2, axis=0, keepdims=True)
        return (x - mu) * jax.lax.rsqrt(var + _LN_EPS)

    ih = jnp.maximum(ln_cols(xt[:128]) * sl[:, 0:1] + sl[:, 1:2], 0.0)
    inter = (jnp.sum(ih * sl[:, 2:3], axis=0, keepdims=True)
             + sl[0:1, 6:7])

    gh = jnp.maximum(ln_cols(xt[128:160]) * sl[:32, 3:4] + sl[:32, 4:5], 0.0)
    gate = (jnp.sum(gh * sl[:32, 5:6], axis=0, keepdims=True)
            + sl[1:2, 6:7])

    weight = inter * jax.nn.sigmoid(gate)

    rvd = rvd_ref[...]
    dist = rvd[4:5, :]
    coef = 5.0 / ((dist + 1e-6) * (dist + 5.0))
    force = jnp.transpose(rvd[:4] * (weight * coef))

    a_mask = jax.lax.broadcasted_iota(jnp.int32, (te, n_hi), 1) == (el >> 7)
    b_hot = (jax.lax.broadcasted_iota(jnp.int32, (te, 128), 1) == (el & 127)
             ).astype(jnp.float32)
    lhs = jnp.concatenate(
        [jnp.where(a_mask, force[:, c:c + 1], 0.0) for c in range(3)],
        axis=1)
    part = jax.lax.dot_general(lhs, b_hot, (((0,), (0,)), ((), ())),
                               preferred_element_type=jnp.float32)
    out_ref[...] += part


def _fin_kernel(part_ref, hn_ref, nx_ref, pw1h_ref, pw1x_ref, slab_ref, out_ref):
    delta = part_ref[0]
    for c in range(1, part_ref.shape[0]):
        delta = delta + part_ref[c]

    sl = slab_ref[...]
    norm = jnp.sqrt(jnp.sum(delta * delta, axis=1, keepdims=True))
    ph = (jnp.dot(hn_ref[...], pw1h_ref[...], preferred_element_type=jnp.float32)
          + jnp.dot(nx_ref[...], pw1x_ref[...], preferred_element_type=jnp.float32)
          + norm * sl[0:1, :] + sl[1:2, :])
    ph = jnp.maximum(_ln_rows(ph) * sl[2:3, :] + sl[3:4, :], 0.0)
    scale = jax.nn.sigmoid(jnp.sum(ph * sl[4:5, :], axis=1, keepdims=True)
                           + sl[5:6, 0:1])
    out_ref[...] = delta * scale


def kernel(h_node, h_edge, edge_index, relative_vec, distance, node_extra,
           lw1, lb1, lg, lbe, lw2, lb2,
           rw1, rb1, rg, rbe, rw2, rb2,
           wb, wn_l, wn_r,
           iw1, ib1, ig, ibe, iw2, ib2,
           gw1_e, gw1_l, gw1_r, gw1_x, gb1, gg, gbe, gw2, gb2,
           pw1_h, pw1_x, pw1_n, pb1, pg, pbe, pw2, pb2,
           *, edge_tile=1024, num_splits=2):
    N, D = h_node.shape
    E, De = h_edge.shape
    N_pad = -(-N // 128) * 128

    f32 = lambda a: jnp.asarray(a, jnp.float32)
    e_left = edge_index[0].astype(jnp.int32)
    e_right = edge_index[1].astype(jnp.int32)

    h_node = f32(h_node)
    nx = f32(node_extra)

    full = lambda a: pl.BlockSpec(a.shape, lambda: (0,) * a.ndim)
    node_ins = [h_node, nx,
                f32(lw1), f32(lb1), f32(lg), f32(lbe), f32(lw2), f32(lb2),
                f32(rw1), f32(rb1), f32(rg), f32(rbe), f32(rw2), f32(rb2),
                f32(wn_l), f32(wn_r), f32(iw1), f32(wb),
                f32(gw1_e), f32(gw1_l), f32(gw1_r), f32(gw1_x),
                f32(ib1), f32(gb1)]
    table_l, table_r, wcomb = pl.pallas_call(
        _node_kernel,
        out_shape=(jax.ShapeDtypeStruct((N, D + 32), jnp.bfloat16),
                   jax.ShapeDtypeStruct((N, D + 32), jnp.bfloat16),
                   jax.ShapeDtypeStruct((De, D + 32), jnp.bfloat16)),
        in_specs=[full(a) for a in node_ins],
        out_specs=(pl.BlockSpec((N, D + 32), lambda: (0, 0)),
                   pl.BlockSpec((N, D + 32), lambda: (0, 0)),
                   pl.BlockSpec((De, D + 32), lambda: (0, 0))),
    )(*node_ins)

    if N_pad > N:
        znp = lambda a: jnp.pad(a, ((0, N_pad - N), (0, 0)))
        table_l, table_r = znp(table_l), znp(table_r)

    he = f32(h_edge)
    rvd = jnp.concatenate(
        [f32(relative_vec).T, jnp.zeros((1, E), jnp.float32),
         f32(distance)[None, :], jnp.zeros((3, E), jnp.float32)], axis=0)

    edge_tile = max(128, (edge_tile // 128) * 128)
    per_split = -(-E // num_splits)
    edge_tile = min(edge_tile, -(-per_split // 128) * 128)
    chunk = num_splits * edge_tile
    E_pad = -(-E // chunk) * chunk
    nk = E_pad // chunk
    pad = E_pad - E
    if pad:
        he = jnp.pad(he, ((0, pad), (0, 0)))
        rvd = jnp.pad(rvd, ((0, 0), (0, pad)))
        e_left = jnp.pad(e_left, (0, pad))
        e_right = jnp.pad(e_right, (0, pad))
    el_col = e_left[:, None]
    er_col = e_right[:, None]

    padc = lambda r: jnp.pad(f32(r).reshape(-1, 1), ((0, 128 - r.size), (0, 0)))
    slab1 = jnp.concatenate(
        [padc(ig), padc(ibe), padc(iw2),
         padc(gg), padc(gbe), padc(gw2),
         padc(jnp.concatenate([f32(ib2).reshape(1, 1),
                               f32(gb2).reshape(1, 1)], axis=0)),
         jnp.zeros((128, 1), jnp.float32)], axis=1)

    estream = lambda cols: pl.BlockSpec((edge_tile, cols),
                                        lambda c, k: (c * nk + k, 0))
    resident = lambda a: pl.BlockSpec(a.shape, lambda c, k: (0, 0))

    partial = pl.pallas_call(
        _edge_kernel,
        out_shape=jax.ShapeDtypeStruct(
            (num_splits, 3 * (N_pad // 128), 128), jnp.float32),
        grid=(num_splits, nk),
        in_specs=[estream(De),
                  pl.BlockSpec((8, edge_tile), lambda c, k: (0, c * nk + k)),
                  estream(1), estream(1),
                  resident(table_l), resident(table_r),
                  resident(wcomb), resident(slab1)],
        out_specs=pl.BlockSpec((None, 3 * (N_pad // 128), 128),
                               lambda c, k: (c, 0, 0)),
        compiler_params=pltpu.CompilerParams(
            dimension_semantics=("parallel", "arbitrary"),
            vmem_limit_bytes=64 * 1024 * 1024),
    )(he, rvd, el_col, er_col, table_l, table_r, wcomb, slab1)

    delta_nm = partial.reshape(num_splits, 3, N_pad).transpose(0, 2, 1)

    npad_n = N_pad - N
    hn_p = jnp.pad(h_node, ((0, npad_n), (0, 0))) if npad_n else h_node
    nx_p = jnp.pad(nx, ((0, npad_n), (0, 0))) if npad_n else nx

    pad128 = lambda r: jnp.pad(f32(r).reshape(1, -1),
                               ((0, 0), (0, 128 - r.size)))
    slab2 = jnp.concatenate(
        [f32(pw1_n), f32(pb1), f32(pg), f32(pbe),
         f32(pw2).reshape(1, -1), pad128(f32(pb2).reshape(1, 1)),
         jnp.zeros((2, 128), jnp.float32)], axis=0)

    fin_ins = [delta_nm, hn_p, nx_p, f32(pw1_h), f32(pw1_x), slab2]
    out = pl.pallas_call(
        _fin_kernel,
        out_shape=jax.ShapeDtypeStruct((N_pad, 3), jnp.float32),
        in_specs=[full(a) for a in fin_ins],
        out_specs=pl.BlockSpec((N_pad, 3), lambda: (0, 0)),
    )(*fin_ins)

    return out[:N]
